# Initial kernel scaffold; baseline (speedup 1.0000x reference)
import jax
import jax.numpy as jnp
from jax.experimental import pallas as pl


def kernel(embedding_weights, sharded_sparse_features):
    raise NotImplementedError("write your pallas kernel here")


# speedup vs baseline: 2.8630x; 2.8630x over previous
import functools

import jax
import jax.numpy as jnp
from jax import lax
from jax.experimental import pallas as pl
from jax.experimental.pallas import tpu as pltpu
from jax.experimental.pallas import tpu_sc as plsc

DIM = 32
BAG = 20
BAGS_PER_CHUNK = 64
ROWS_PER_CHUNK = BAGS_PER_CHUNK * BAG
IDX_MINOR = 128
IDX_ROWS = ROWS_PER_CHUNK // IDX_MINOR


@functools.partial(jax.jit, static_argnums=(2, 3))
def _sc_lookup(table, idx3d, total_chunks, chunks_per_worker):
    mesh = plsc.VectorSubcoreMesh(core_axis_name="c", subcore_axis_name="s")

    @functools.partial(
        pl.kernel,
        mesh=mesh,
        compiler_params=pltpu.CompilerParams(use_tc_tiling_on_sc=False),
        out_type=jax.ShapeDtypeStruct(
            (total_chunks, BAGS_PER_CHUNK, DIM), jnp.float32
        ),
        scratch_types=[
            pltpu.VMEM((2, IDX_ROWS, IDX_MINOR), jnp.int32),
            pltpu.VMEM((2, ROWS_PER_CHUNK, DIM), jnp.float32),
            pltpu.VMEM((BAGS_PER_CHUNK, DIM), jnp.float32),
            pltpu.SemaphoreType.DMA,
            pltpu.SemaphoreType.DMA,
        ],
    )
    def k(table_hbm, idx_hbm, out_hbm, idx_v, rows_v, out_v, sem0, sem1):
        sems = (sem0, sem1)
        wid = lax.axis_index("s") * 2 + lax.axis_index("c")
        c0 = wid * chunks_per_worker

        def start(gc, b):
            pltpu.sync_copy(idx_hbm.at[gc], idx_v.at[b])
            for j in range(IDX_ROWS):
                pltpu.async_copy(
                    table_hbm.at[idx_v.at[b, j]],
                    rows_v.at[b, pl.ds(j * IDX_MINOR, IDX_MINOR)],
                    sems[b],
                )

        def wait(b):
            for j in range(IDX_ROWS):
                pltpu.make_async_copy(
                    table_hbm.at[idx_v.at[b, j]],
                    rows_v.at[b, pl.ds(j * IDX_MINOR, IDX_MINOR)],
                    sems[b],
                ).wait()

        def reduce_store(gc, b):
            def bag_body(bag, carry):
                base = bag * BAG
                a0 = rows_v[b, base, pl.ds(0, 16)]
                a1 = rows_v[b, base, pl.ds(16, 16)]
                for l in range(1, BAG):
                    a0 = a0 + rows_v[b, base + l, pl.ds(0, 16)]
                    a1 = a1 + rows_v[b, base + l, pl.ds(16, 16)]
                out_v[bag, pl.ds(0, 16)] = a0
                out_v[bag, pl.ds(16, 16)] = a1
                return carry

            lax.fori_loop(0, BAGS_PER_CHUNK, bag_body, 0)
            pltpu.sync_copy(out_v, out_hbm.at[gc])

        start(c0, 0)

        def pair_body(g, carry):
            for bpar in range(2):
                c = 2 * g + bpar
                nxt = 1 - bpar

                @pl.when(c + 1 < chunks_per_worker)
                def _():
                    start(c0 + c + 1, nxt)

                wait(bpar)
                reduce_store(c0 + c, bpar)
            return carry

        lax.fori_loop(0, chunks_per_worker // 2, pair_body, 0)

    return k(table, idx3d)


def kernel(embedding_weights, sharded_sparse_features):
    N, T, D = embedding_weights.shape
    B, _, L = sharded_sparse_features.shape
    table = embedding_weights.reshape(N * T, D)
    flat_idx = sharded_sparse_features.astype(jnp.int32) * T + jnp.arange(
        T, dtype=jnp.int32
    )[None, :, None]
    total_rows = B * T * L
    total_chunks = total_rows // ROWS_PER_CHUNK
    num_workers = 32
    chunks_per_worker = total_chunks // num_workers
    idx3d = flat_idx.reshape(total_chunks, IDX_ROWS, IDX_MINOR)
    out = _sc_lookup(table, idx3d, total_chunks, chunks_per_worker)
    return out.reshape(B, T, D)


# speedup vs baseline: 2.9378x; 1.0261x over previous
import functools

import jax
import jax.numpy as jnp
from jax import lax
from jax.experimental import pallas as pl
from jax.experimental.pallas import tpu as pltpu
from jax.experimental.pallas import tpu_sc as plsc

DIM = 32
BAG = 20
BC = 64
ROWS = BAG * BC
IDX_MINOR = 128
IDX_ROWS = ROWS // IDX_MINOR


@functools.partial(jax.jit, static_argnums=(2, 3, 4))
def _sc_lookup(table, idx_t, T, B, num_workers):
    units_per_worker = (T * B // BC) // num_workers
    mesh = plsc.VectorSubcoreMesh(core_axis_name="c", subcore_axis_name="s")

    @functools.partial(
        pl.kernel,
        mesh=mesh,
        compiler_params=pltpu.CompilerParams(
            use_tc_tiling_on_sc=False, needs_layout_passes=False
        ),
        out_type=jax.ShapeDtypeStruct((T, DIM, B), jnp.float32),
        scratch_types=[
            pltpu.VMEM((2, BAG, BC), jnp.int32),
            pltpu.VMEM((2, IDX_ROWS, IDX_MINOR), jnp.int32),
            pltpu.VMEM((2, ROWS, DIM), jnp.float32),
            pltpu.VMEM((DIM, BC), jnp.float32),
            pltpu.SemaphoreType.DMA,
            pltpu.SemaphoreType.DMA,
        ],
    )
    def k(table_hbm, idx_hbm, out_hbm, idx_v, flat_v, rows_v, out_v, sem0, sem1):
        sems = (sem0, sem1)
        wid = lax.axis_index("s") * 2 + lax.axis_index("c")
        b0s = (wid * 2 * BC, (wid * 2 + 1) * BC)

        def start(t, sub, b):
            pltpu.sync_copy(idx_hbm.at[t, :, pl.ds(b0s[sub], BC)], idx_v.at[b])
            for kk in range(ROWS // 16):
                v = idx_v[b, kk // 4, pl.ds((kk % 4) * 16, 16)]
                flat_v[b, kk // 8, pl.ds((kk % 8) * 16, 16)] = v * T + t
            for j in range(IDX_ROWS):
                pltpu.async_copy(
                    table_hbm.at[flat_v.at[b, j]],
                    rows_v.at[b, pl.ds(j * IDX_MINOR, IDX_MINOR)],
                    sems[b],
                )

        def wait(b):
            for j in range(IDX_ROWS):
                pltpu.make_async_copy(
                    table_hbm.at[flat_v.at[b, j]],
                    rows_v.at[b, pl.ds(j * IDX_MINOR, IDX_MINOR)],
                    sems[b],
                ).wait()

        lane = jax.lax.iota(jnp.int32, 16)
        row_lo = lane
        row_hi = lane + 16

        def reduce_store(t, sub, b):
            def bag_body(bag, carry):
                a0 = rows_v[b, bag, pl.ds(0, 16)]
                a1 = rows_v[b, bag, pl.ds(16, 16)]
                for l in range(1, BAG):
                    a0 = a0 + rows_v[b, bag + l * BC, pl.ds(0, 16)]
                    a1 = a1 + rows_v[b, bag + l * BC, pl.ds(16, 16)]
                col = jnp.full((16,), 0, jnp.int32) + bag
                plsc.store_scatter(out_v, [row_lo, col], a0)
                plsc.store_scatter(out_v, [row_hi, col], a1)
                return carry

            lax.fori_loop(0, BC, bag_body, 0)
            pltpu.sync_copy(out_v, out_hbm.at[t, :, pl.ds(b0s[sub], BC)])

        start(0, 0, 0)

        def pair_body(g, carry):
            for bpar in range(2):
                u = 2 * g + bpar
                nxt = 1 - bpar

                @pl.when(u + 1 < units_per_worker)
                def _():
                    start(g + bpar, nxt, nxt)

                wait(bpar)
                reduce_store(g, bpar, bpar)
            return carry

        lax.fori_loop(0, units_per_worker // 2, pair_body, 0)

    return k(table, idx_t)


def kernel(embedding_weights, sharded_sparse_features):
    N, T, D = embedding_weights.shape
    B, _, L = sharded_sparse_features.shape
    table = embedding_weights.reshape(N * T, D)
    idx_t = sharded_sparse_features.astype(jnp.int32).transpose(1, 2, 0)
    out = _sc_lookup(table, idx_t, T, B, 32)
    return out.transpose(2, 0, 1)
